# baseline (speedup 1.0000x reference)
import functools

import jax
import jax.numpy as jnp
from jax import lax
from jax.experimental import pallas as pl
from jax.experimental.pallas import tpu as pltpu
from jax.experimental.pallas import tpu_sc as plsc

N = 10000
E = 320000
D = 128

NC = 2
NS = 16
NW = NC * NS
CHUNK = 128
NBUF = 2
CPW = 80
HCPW = CPW // 2
EPAD = NW * CPW * CHUNK
NPAD = 10112
RPT = NPAD // NS
DEGW = 8

BLK = 1000
GRID = N // BLK

_HIGHEST = jax.lax.Precision.HIGHEST


def _sc_mesh():
    return plsc.VectorSubcoreMesh(core_axis_name="c", subcore_axis_name="s")


def _deg_body(dstp_hbm, ones_hbm, zeros_hbm, out_hbm, dst_v, ones_v, deg_sh):
    cid = lax.axis_index("c")
    sid = lax.axis_index("s")
    wid = cid * NS + sid
    pltpu.sync_copy(zeros_hbm, deg_sh.at[pl.ds(sid * RPT, RPT)])
    pltpu.sync_copy(ones_hbm, ones_v)
    pltpu.sync_copy(dstp_hbm.at[wid], dst_v)
    plsc.subcore_barrier()

    def body(j, carry):
        pltpu.sync_copy(ones_v, deg_sh.at[dst_v.at[j]], add=True)
        return carry

    lax.fori_loop(0, CPW, body, 0)
    plsc.subcore_barrier()
    pltpu.sync_copy(deg_sh.at[pl.ds(sid * RPT, RPT)],
                    out_hbm.at[cid, pl.ds(sid * RPT, RPT)])


_deg_call = functools.partial(
    pl.kernel,
    out_type=jax.ShapeDtypeStruct((NC, NPAD, DEGW), jnp.float32),
    mesh=_sc_mesh(),
    scratch_types=[
        pltpu.VMEM((CPW, CHUNK), jnp.int32),
        pltpu.VMEM((CHUNK, DEGW), jnp.float32),
        pltpu.VMEM_SHARED((NPAD, DEGW), jnp.float32),
    ],
)(_deg_body)


def _agg_body(h_hbm, srcp_hbm, dstp_hbm, zeros_hbm, out_hbm,
              src_v, dst_v, rows0, rows1, acc_sh,
              g0, g1):
    cid = lax.axis_index("c")
    sid = lax.axis_index("s")
    wid = cid * NS + sid
    bufs = (rows0, rows1)
    gsems = (g0, g1)
    pltpu.sync_copy(zeros_hbm, acc_sh.at[pl.ds(sid * RPT, RPT)])
    plsc.subcore_barrier()

    for half in range(2):
        pltpu.sync_copy(srcp_hbm.at[wid, pl.ds(half * HCPW, HCPW)], src_v)
        pltpu.sync_copy(dstp_hbm.at[wid, pl.ds(half * HCPW, HCPW)], dst_v)

        def body(j, carry):
            pltpu.async_copy(h_hbm.at[src_v.at[j]], bufs[0], gsems[0]).wait()
            pltpu.sync_copy(bufs[0], acc_sh.at[dst_v.at[j]], add=True)
            return carry

        lax.fori_loop(0, HCPW, body, 0)
    plsc.subcore_barrier()
    pltpu.sync_copy(acc_sh.at[pl.ds(sid * RPT, RPT)],
                    out_hbm.at[cid, pl.ds(sid * RPT, RPT)])


_agg_call = functools.partial(
    pl.kernel,
    out_type=jax.ShapeDtypeStruct((NC, NPAD, D), jnp.float32),
    mesh=_sc_mesh(),
    scratch_types=(
        [pltpu.VMEM((HCPW, CHUNK), jnp.int32),
         pltpu.VMEM((HCPW, CHUNK), jnp.int32)]
        + [pltpu.VMEM((CHUNK, D), jnp.float32)] * NBUF
        + [pltpu.VMEM_SHARED((NPAD, D), jnp.float32)]
        + [pltpu.SemaphoreType.DMA] * NBUF
    ),
)(_agg_body)


def _tc0_body(x_ref, degp_ref, tab_ref, w0_ref, h0p_ref, dinv_ref):
    deg = degp_ref[0, 0] + degp_ref[1, 0] + 1.0
    dinv = lax.rsqrt(deg)
    dinv_ref[0] = dinv
    xv = x_ref[0]
    z = jnp.zeros((BLK, D), jnp.float32)
    for t in range(6):
        z = z + jnp.where(xv == t, tab_ref[t:t + 1, :], 0.0)
    h0 = jnp.dot(z, w0_ref[...], preferred_element_type=jnp.float32,
                 precision=_HIGHEST)
    h0p_ref[...] = dinv * h0


def _tc1_body(acc_ref, hp_ref, dinv_ref, w_ref, b_ref, out_ref):
    dinv = dinv_ref[0]
    s = acc_ref[0] + acc_ref[1] + hp_ref[...]
    z1 = jnp.maximum(dinv * s + b_ref[...], 0.0)
    h1 = jnp.dot(z1, w_ref[...], preferred_element_type=jnp.float32,
                 precision=_HIGHEST)
    out_ref[...] = dinv * h1


def _tc2_body(acc_ref, hp_ref, dinv_ref, b_ref, pw1_ref, pb1_ref,
              pw2_ref, pb2_ref, z2_ref, proj_ref):
    dinv = dinv_ref[0]
    s = acc_ref[0] + acc_ref[1] + hp_ref[...]
    z2 = jnp.maximum(dinv * s + b_ref[...], 0.0)
    z2_ref[...] = z2
    p1 = jnp.maximum(
        jnp.dot(z2, pw1_ref[...], preferred_element_type=jnp.float32,
                precision=_HIGHEST) + pb1_ref[...], 0.0)
    proj_ref[...] = (
        jnp.dot(p1, pw2_ref[...], preferred_element_type=jnp.float32,
                precision=_HIGHEST) + pb2_ref[...])


def _full(shape):
    return pl.BlockSpec(shape, lambda i: tuple(0 for _ in shape))


def _tc0_call(x3, degp4, tab, w0):
    return pl.pallas_call(
        _tc0_body,
        grid=(GRID,),
        in_specs=[
            pl.BlockSpec((1, BLK, 1), lambda i: (i, 0, 0)),
            pl.BlockSpec((2, 1, BLK, 1), lambda i: (0, i, 0, 0)),
            _full((8, D)),
            _full((D, D)),
        ],
        out_specs=[
            pl.BlockSpec((BLK, D), lambda i: (i, 0)),
            pl.BlockSpec((1, BLK, 1), lambda i: (i, 0, 0)),
        ],
        out_shape=[
            jax.ShapeDtypeStruct((N, D), jnp.float32),
            jax.ShapeDtypeStruct((GRID, BLK, 1), jnp.float32),
        ],
    )(x3, degp4, tab, w0)


def _tc1_call(acc, hp, dinv3, w, b2):
    return pl.pallas_call(
        _tc1_body,
        grid=(GRID,),
        in_specs=[
            pl.BlockSpec((2, BLK, D), lambda i: (0, i, 0)),
            pl.BlockSpec((BLK, D), lambda i: (i, 0)),
            pl.BlockSpec((1, BLK, 1), lambda i: (i, 0, 0)),
            _full((D, D)),
            _full((1, D)),
        ],
        out_specs=pl.BlockSpec((BLK, D), lambda i: (i, 0)),
        out_shape=jax.ShapeDtypeStruct((N, D), jnp.float32),
    )(acc, hp, dinv3, w, b2)


def _tc2_call(acc, hp, dinv3, b2, pw1, pb1, pw2, pb2):
    return pl.pallas_call(
        _tc2_body,
        grid=(GRID,),
        in_specs=[
            pl.BlockSpec((2, BLK, D), lambda i: (0, i, 0)),
            pl.BlockSpec((BLK, D), lambda i: (i, 0)),
            pl.BlockSpec((1, BLK, 1), lambda i: (i, 0, 0)),
            _full((1, D)),
            _full((D, D)),
            _full((1, D)),
            _full((D, D)),
            _full((1, D)),
        ],
        out_specs=[
            pl.BlockSpec((BLK, D), lambda i: (i, 0)),
            pl.BlockSpec((BLK, D), lambda i: (i, 0)),
        ],
        out_shape=[
            jax.ShapeDtypeStruct((N, D), jnp.float32),
            jax.ShapeDtypeStruct((N, D), jnp.float32),
        ],
    )(acc, hp, dinv3, b2, pw1, pb1, pw2, pb2)


def kernel(x, edge_index, node_type_embed, edge_type_embed,
           W0, b0, W1, b1, pW1, pb1, pW2, pb2):
    del edge_type_embed

    src = edge_index[0]
    dst = edge_index[1]
    pad = EPAD - E
    srcp = jnp.concatenate([src, jnp.zeros((pad,), jnp.int32)])
    dstp = jnp.concatenate([dst, jnp.full((pad,), N, jnp.int32)])
    srcp = srcp.reshape(NW, CPW, CHUNK)
    dstp = dstp.reshape(NW, CPW, CHUNK)

    ones8 = jnp.ones((CHUNK, DEGW), jnp.float32)
    zeros8 = jnp.zeros((RPT, DEGW), jnp.float32)
    zerosD = jnp.zeros((RPT, D), jnp.float32)
    tab8 = jnp.zeros((8, D), jnp.float32).at[:6].set(node_type_embed)
    x3 = x.reshape(GRID, BLK, 1)
    b0r = b0.reshape(1, D)
    b1r = b1.reshape(1, D)
    pb1r = pb1.reshape(1, D)
    pb2r = pb2.reshape(1, D)

    degp = _deg_call(dstp, ones8, zeros8)
    degp4 = degp[:, :N, 0].reshape(2, GRID, BLK, 1)

    h0p, dinv3 = _tc0_call(x3, degp4, tab8, W0)

    acc0 = _agg_call(h0p, srcp, dstp, zerosD)

    h1p = _tc1_call(acc0, h0p, dinv3, W1, b0r)

    acc1 = _agg_call(h1p, srcp, dstp, zerosD)

    z2, proj = _tc2_call(acc1, h1p, dinv3, b1r, pW1, pb1r, pW2, pb2r)
    return (z2, proj)


# speedup vs baseline: 1.4902x; 1.4902x over previous
import functools

import jax
import jax.numpy as jnp
from jax import lax
from jax.experimental import pallas as pl
from jax.experimental.pallas import tpu as pltpu
from jax.experimental.pallas import tpu_sc as plsc

N = 10000
E = 320000
D = 128

NC = 2
NS = 16
NW = NC * NS
CHUNK = 128
CPW = 79
EPILOG = 0
EPAD = NW * CPW * CHUNK
NPAD = 10240
RPT = NPAD // NS
DEGW = 8

BLK = 1000
GRID = N // BLK

_HIGHEST = jax.lax.Precision.HIGHEST


def _sc_mesh():
    return plsc.VectorSubcoreMesh(core_axis_name="c", subcore_axis_name="s")


def _deg_body(dstp_hbm, ones_hbm, zeros_hbm, out_hbm, dst_v, ones_v, deg_sh):
    cid = lax.axis_index("c")
    sid = lax.axis_index("s")
    wid = cid * NS + sid
    pltpu.sync_copy(zeros_hbm, deg_sh.at[pl.ds(sid * RPT, RPT)])
    pltpu.sync_copy(ones_hbm, ones_v)
    pltpu.sync_copy(dstp_hbm.at[wid], dst_v)
    plsc.subcore_barrier()

    def body(j, carry):
        pltpu.sync_copy(ones_v, deg_sh.at[dst_v.at[j]], add=True)
        return carry

    lax.fori_loop(0, CPW, body, 0)
    plsc.subcore_barrier()
    pltpu.sync_copy(deg_sh.at[pl.ds(sid * RPT, RPT)],
                    out_hbm.at[cid, pl.ds(sid * RPT, RPT)])


_deg_call = functools.partial(
    pl.kernel,
    out_type=jax.ShapeDtypeStruct((NC, NPAD, DEGW), jnp.float32),
    mesh=_sc_mesh(),
    scratch_types=[
        pltpu.VMEM((CPW, CHUNK), jnp.int32),
        pltpu.VMEM((CHUNK, DEGW), jnp.float32),
        pltpu.VMEM_SHARED((NPAD, DEGW), jnp.float32),
    ],
)(_deg_body)


def _agg_body(h_hbm, srcp_hbm, dstp_hbm, zeros_hbm, out_hbm,
              src_v, dst_v, rows_v, acc_sh, gsem):
    cid = lax.axis_index("c")
    sid = lax.axis_index("s")
    wid = cid * NS + sid
    pltpu.sync_copy(zeros_hbm, acc_sh.at[pl.ds(sid * RPT, RPT)])
    pltpu.sync_copy(srcp_hbm.at[wid], src_v)
    pltpu.sync_copy(dstp_hbm.at[wid], dst_v)
    plsc.subcore_barrier()

    def body(j, carry):
        pltpu.async_copy(h_hbm.at[src_v.at[j]], rows_v, gsem).wait()
        pltpu.sync_copy(rows_v, acc_sh.at[dst_v.at[j]], add=True)
        return carry

    lax.fori_loop(0, CPW, body, 0)
    plsc.subcore_barrier()
    pltpu.sync_copy(acc_sh.at[pl.ds(sid * RPT, RPT)],
                    out_hbm.at[cid, pl.ds(sid * RPT, RPT)])


_agg_call = functools.partial(
    pl.kernel,
    out_type=jax.ShapeDtypeStruct((NC, NPAD, D), jnp.float32),
    mesh=_sc_mesh(),
    scratch_types=[
        pltpu.VMEM((CPW, CHUNK), jnp.int32),
        pltpu.VMEM((CPW, CHUNK), jnp.int32),
        pltpu.VMEM((CHUNK, D), jnp.float32),
        pltpu.VMEM_SHARED((NPAD, D), jnp.float32),
        pltpu.SemaphoreType.DMA,
    ],
)(_agg_body)


def _tc0_body(x_ref, degp_ref, tab_ref, w0_ref, h0p_ref, dinv_ref):
    deg = degp_ref[0, 0] + degp_ref[1, 0] + 1.0
    dinv = lax.rsqrt(deg)
    dinv_ref[0] = dinv
    xv = x_ref[0]
    z = jnp.zeros((BLK, D), jnp.float32)
    for t in range(6):
        z = z + jnp.where(xv == t, tab_ref[t:t + 1, :], 0.0)
    h0 = jnp.dot(z, w0_ref[...], preferred_element_type=jnp.float32,
                 precision=_HIGHEST)
    h0p_ref[...] = dinv * h0


def _tc1_body(acc_ref, hp_ref, dinv_ref, w_ref, b_ref, out_ref):
    dinv = dinv_ref[0]
    s = acc_ref[0] + acc_ref[1] + hp_ref[...]
    z1 = jnp.maximum(dinv * s + b_ref[...], 0.0)
    h1 = jnp.dot(z1, w_ref[...], preferred_element_type=jnp.float32,
                 precision=_HIGHEST)
    out_ref[...] = dinv * h1


def _tc2_body(acc_ref, hp_ref, dinv_ref, b_ref, pw1_ref, pb1_ref,
              pw2_ref, pb2_ref, z2_ref, proj_ref):
    dinv = dinv_ref[0]
    s = acc_ref[0] + acc_ref[1] + hp_ref[...]
    z2 = jnp.maximum(dinv * s + b_ref[...], 0.0)
    z2_ref[...] = z2
    p1 = jnp.maximum(
        jnp.dot(z2, pw1_ref[...], preferred_element_type=jnp.float32,
                precision=_HIGHEST) + pb1_ref[...], 0.0)
    proj_ref[...] = (
        jnp.dot(p1, pw2_ref[...], preferred_element_type=jnp.float32,
                precision=_HIGHEST) + pb2_ref[...])


def _full(shape):
    return pl.BlockSpec(shape, lambda i: tuple(0 for _ in shape))


def _tc0_call(x3, degp4, tab, w0):
    return pl.pallas_call(
        _tc0_body,
        grid=(GRID,),
        in_specs=[
            pl.BlockSpec((1, BLK, 1), lambda i: (i, 0, 0)),
            pl.BlockSpec((2, 1, BLK, 1), lambda i: (0, i, 0, 0)),
            _full((8, D)),
            _full((D, D)),
        ],
        out_specs=[
            pl.BlockSpec((BLK, D), lambda i: (i, 0)),
            pl.BlockSpec((1, BLK, 1), lambda i: (i, 0, 0)),
        ],
        out_shape=[
            jax.ShapeDtypeStruct((N, D), jnp.float32),
            jax.ShapeDtypeStruct((GRID, BLK, 1), jnp.float32),
        ],
    )(x3, degp4, tab, w0)


def _tc1_call(acc, hp, dinv3, w, b2):
    return pl.pallas_call(
        _tc1_body,
        grid=(GRID,),
        in_specs=[
            pl.BlockSpec((2, BLK, D), lambda i: (0, i, 0)),
            pl.BlockSpec((BLK, D), lambda i: (i, 0)),
            pl.BlockSpec((1, BLK, 1), lambda i: (i, 0, 0)),
            _full((D, D)),
            _full((1, D)),
        ],
        out_specs=pl.BlockSpec((BLK, D), lambda i: (i, 0)),
        out_shape=jax.ShapeDtypeStruct((N, D), jnp.float32),
    )(acc, hp, dinv3, w, b2)


def _tc2_call(acc, hp, dinv3, b2, pw1, pb1, pw2, pb2):
    return pl.pallas_call(
        _tc2_body,
        grid=(GRID,),
        in_specs=[
            pl.BlockSpec((2, BLK, D), lambda i: (0, i, 0)),
            pl.BlockSpec((BLK, D), lambda i: (i, 0)),
            pl.BlockSpec((1, BLK, 1), lambda i: (i, 0, 0)),
            _full((1, D)),
            _full((D, D)),
            _full((1, D)),
            _full((D, D)),
            _full((1, D)),
        ],
        out_specs=[
            pl.BlockSpec((BLK, D), lambda i: (i, 0)),
            pl.BlockSpec((BLK, D), lambda i: (i, 0)),
        ],
        out_shape=[
            jax.ShapeDtypeStruct((N, D), jnp.float32),
            jax.ShapeDtypeStruct((N, D), jnp.float32),
        ],
    )(acc, hp, dinv3, b2, pw1, pb1, pw2, pb2)


def kernel(x, edge_index, node_type_embed, edge_type_embed,
           W0, b0, W1, b1, pW1, pb1, pW2, pb2):
    del edge_type_embed

    src = edge_index[0]
    dst = edge_index[1]
    pad = EPAD - E
    srcp = jnp.concatenate([src, jnp.zeros((pad,), jnp.int32)])
    dstp = jnp.concatenate([dst, jnp.full((pad,), N, jnp.int32)])
    srcp = srcp.reshape(NW, CPW, CHUNK)
    dstp = dstp.reshape(NW, CPW, CHUNK)

    ones8 = jnp.ones((CHUNK, DEGW), jnp.float32)
    zeros8 = jnp.zeros((RPT, DEGW), jnp.float32)
    zerosD = jnp.zeros((RPT, D), jnp.float32)
    tab8 = jnp.zeros((8, D), jnp.float32).at[:6].set(node_type_embed)
    x3 = x.reshape(GRID, BLK, 1)
    b0r = b0.reshape(1, D)
    b1r = b1.reshape(1, D)
    pb1r = pb1.reshape(1, D)
    pb2r = pb2.reshape(1, D)

    degp = _deg_call(dstp, ones8, zeros8)
    degp4 = degp[:, :N, 0].reshape(2, GRID, BLK, 1)

    h0p, dinv3 = _tc0_call(x3, degp4, tab8, W0)

    acc0 = _agg_call(h0p, srcp, dstp, zerosD)

    h1p = _tc1_call(acc0, h0p, dinv3, W1, b0r)

    acc1 = _agg_call(h1p, srcp, dstp, zerosD)

    z2, proj = _tc2_call(acc1, h1p, dinv3, b1r, pW1, pb1r, pW2, pb2r)
    return (z2, proj)
